# baseline (speedup 1.0000x reference)
import functools

import jax
import jax.numpy as jnp
from jax import lax
from jax.experimental import pallas as pl
from jax.experimental.pallas import tpu as pltpu
from jax.experimental.pallas import tpu_sc as plsc

_N = 50000
_NU = 25000
_E = 800000
_NP = 51200
_K = 128
_GB = 4
_NSUB = 16
_NCORE = 2
_NRR = _NP // _NSUB
_BN = 3200


def _ceil_to(x, m):
    return (x + m - 1) // m * m


_EP = _ceil_to(_E, _K * _GB * _NSUB)
_CPB = _EP // _K


@functools.lru_cache(None)
def _make_seg_sum(nb, w, shared=False):
    gb = _GB if w == 16 else 2
    ch = nb // _NCORE
    cpt = _CPB // _NSUB
    ngrp = cpt // gb
    mesh = plsc.VectorSubcoreMesh(core_axis_name="c", subcore_axis_name="s")

    def body(vals, sd, zrow, out, acc, sdix, rows, gsem, ssem):
        c = lax.axis_index("c")
        s = lax.axis_index("s")

        def load(b, g, slot):
            sdi = 2 if shared else b
            pltpu.sync_copy(sd.at[sdi, pl.ds(s * cpt + g * gb, gb)],
                            sdix.at[slot])
            for j in range(gb):
                for t in range(_K // 16):
                    ix = pl.ds(t * 16, 16)
                    sdix[slot, j, 0, ix] = sdix[slot, j, 0, ix] * nb + b

        def fire_gathers(slot):
            for j in range(gb):
                pltpu.async_copy(vals.at[sdix.at[slot, j, 0]],
                                 rows.at[slot, j], gsem)

        def wait_gathers(slot):
            for j in range(gb):
                pltpu.make_async_copy(vals.at[sdix.at[slot, j, 0]],
                                      rows.at[slot, j], gsem).wait()

        def scatters(slot):
            descs = [
                pltpu.async_copy(rows.at[slot, j], acc.at[sdix.at[slot, j, 1]],
                                 ssem, add=True)
                for j in range(gb)
            ]
            for d in descs:
                d.wait()

        def step(b, g, p, prefetch, gnext):
            pn = 1 - p
            if prefetch:
                load(b, gnext, pn)
            wait_gathers(p)
            if prefetch:
                fire_gathers(pn)
            scatters(p)

        for bi in range(ch):
            b = c * ch + bi
            pltpu.sync_copy(zrow, rows.at[0, 0])
            zd = [pltpu.async_copy(rows.at[0, 0],
                                   acc.at[pl.ds(s * _NRR + r * _K, _K)], gsem)
                  for r in range(_NRR // _K)]
            for d in zd:
                d.wait()
            plsc.subcore_barrier()

            load(b, 0, 0)
            fire_gathers(0)
            m = (ngrp - 1) // 2

            def dbl(t, carry):
                g = t * 2
                step(b, g, 0, True, g + 1)
                step(b, g + 1, 1, True, g + 2)
                return carry

            lax.fori_loop(0, m, dbl, 0)
            g0 = 2 * m
            if (ngrp - 1) % 2 == 1:
                step(b, g0, 0, True, g0 + 1)
                step(b, g0 + 1, 1, False, 0)
            else:
                step(b, g0, 0, False, 0)
            plsc.subcore_barrier()
            pltpu.sync_copy(acc.at[pl.ds(s * _NRR, _NRR)],
                            out.at[pl.ds(s * _NRR, _NRR), b])
            plsc.subcore_barrier()

    return pl.kernel(
        body,
        out_type=jax.ShapeDtypeStruct((_NP, nb, w), jnp.float32),
        mesh=mesh,
        scratch_types=[
            pltpu.VMEM_SHARED((_NP, w), jnp.float32),
            pltpu.VMEM((2, gb, 2, _K), jnp.int32),
            pltpu.VMEM((2, gb, _K, w), jnp.float32),
            pltpu.SemaphoreType.DMA,
            pltpu.SemaphoreType.DMA,
        ],
        compiler_params=pltpu.CompilerParams(use_tc_tiling_on_sc=False),
    )


@functools.lru_cache(None)
def _make_hist():
    gb = _GB
    cpt_f = _CPB // _NSUB
    cpt_h = _CPB // 2 // _NSUB
    ngrp_f = cpt_f // gb
    ngrp_h = cpt_h // gb
    mesh = plsc.VectorSubcoreMesh(core_axis_name="c", subcore_axis_name="s")

    def body(ones, sd, zrow, out, acc, sdix, rows, gsem, ssem):
        c = lax.axis_index("c")
        s = lax.axis_index("s")
        for j in range(gb):
            pltpu.sync_copy(ones, rows.at[0, j])
        pltpu.sync_copy(zrow, rows.at[1, 0])

        for bi in range(2):
            b = c * 2 + bi
            sdi = 2 * c if bi == 0 else 1 + c
            base0 = jnp.where(c == 0, s * cpt_f,
                              bi * (_CPB // 2) + s * cpt_h)
            ngrp = jnp.where(c == 0, ngrp_f, ngrp_h)
            zd = [pltpu.async_copy(rows.at[1, 0],
                                   acc.at[pl.ds(s * _NRR + r * _K, _K)], gsem)
                  for r in range(_NRR // _K)]
            for d in zd:
                d.wait()
            plsc.subcore_barrier()

            def grp(g, carry):
                pltpu.sync_copy(sd.at[sdi, pl.ds(base0 + g * gb, gb)],
                                sdix.at[0])
                descs = [
                    pltpu.async_copy(rows.at[0, j],
                                     acc.at[sdix.at[0, j, 1]], ssem, add=True)
                    for j in range(gb)
                ]
                for d in descs:
                    d.wait()
                return carry

            lax.fori_loop(0, ngrp, grp, 0)
            plsc.subcore_barrier()
            pltpu.sync_copy(acc.at[pl.ds(s * _NRR, _NRR)],
                            out.at[pl.ds(s * _NRR, _NRR), b])
            plsc.subcore_barrier()

    return pl.kernel(
        body,
        out_type=jax.ShapeDtypeStruct((_NP, 4, 16), jnp.float32),
        mesh=mesh,
        scratch_types=[
            pltpu.VMEM_SHARED((_NP, 16), jnp.float32),
            pltpu.VMEM((2, _GB, 2, _K), jnp.int32),
            pltpu.VMEM((2, _GB, _K, 16), jnp.float32),
            pltpu.SemaphoreType.DMA,
            pltpu.SemaphoreType.DMA,
        ],
        compiler_params=pltpu.CompilerParams(use_tc_tiling_on_sc=False),
    )


def _tc_run(body, row_ins, bcast_ins, out_cols):
    grid = (_NP // _BN,)
    in_specs = (
        [pl.BlockSpec((_BN, a.shape[1]), lambda i: (i, 0)) for a in row_ins]
        + [pl.BlockSpec(w.shape, lambda i: (0, 0)) for w in bcast_ins]
    )
    out_specs = [pl.BlockSpec((_BN, c), lambda i: (i, 0)) for c in out_cols]
    out_shape = [jax.ShapeDtypeStruct((_NP, c), jnp.float32) for c in out_cols]
    return pl.pallas_call(
        body, grid=grid, in_specs=in_specs, out_specs=out_specs,
        out_shape=out_shape,
    )(*row_ins, *bcast_ins)


def _row_mask(cols):
    rid = (pl.program_id(0) * _BN
           + lax.broadcasted_iota(jnp.int32, (_BN, cols), 0))
    return rid < _NU


def _dinvs(h_r):
    diu = 1.0 / jnp.sqrt(h_r[:, 0:1] + 1.0)
    div = 1.0 / jnp.sqrt(h_r[:, 16:17] + 1.0)
    diuv = 1.0 / jnp.sqrt(h_r[:, 32:33] + h_r[:, 48:49] + 1.0)
    return diu, div, diuv


def _t1_body(x_r, h_r, xsa_r):
    diu, div, _ = _dinvs(h_r)
    x = x_r[...]
    xsa_r[...] = jnp.concatenate(
        [jnp.broadcast_to(x * diu, (_BN, 16)),
         jnp.broadcast_to(x * div, (_BN, 16))], axis=1)


def _t2_body(agg_r, xsa_r, h_r, wu_r, bu_r, wv_r, bv_r, xsb_r):
    diu, div, diuv = _dinvs(h_r)
    aggu0 = agg_r[:, 0:1] + xsa_r[:, 0:1]
    aggv0 = agg_r[:, 16:17] + xsa_r[:, 16:17]
    yu = bu_r[...] + (diu * aggu0) * wu_r[...]
    yv = bv_r[...] + (div * aggv0) * wv_r[...]
    xsb_r[...] = diuv * jnp.where(_row_mask(64), yu, yv)


def _t3_body(agg_r, xsb_r, h_r, wuv_r, buv_r, wu2_r, wv2_r, xsc_r):
    diu, div, diuv = _dinvs(h_r)
    x2 = jax.nn.relu(
        buv_r[...]
        + jnp.dot(diuv * (agg_r[...] + xsb_r[...]), wuv_r[...],
                  preferred_element_type=jnp.float32))
    xsc_r[...] = jnp.concatenate(
        [diu * jnp.dot(x2, wu2_r[...], preferred_element_type=jnp.float32),
         div * jnp.dot(x2, wv2_r[...], preferred_element_type=jnp.float32)],
        axis=1)


def _t4_body(agg_r, xsc_r, h_r, bu_r, bv_r, xsd_r):
    diu, div, diuv = _dinvs(h_r)
    outu = bu_r[...] + diu * (agg_r[:, 0:32] + xsc_r[:, 0:32])
    outv = bv_r[...] + div * (agg_r[:, 32:64] + xsc_r[:, 32:64])
    xsd_r[...] = diuv * jnp.where(_row_mask(32), outu, outv)


def _t5_body(agg_r, xsd_r, h_r, wuv_r, buv_r, xse_r):
    diu, div, diuv = _dinvs(h_r)
    x4 = jax.nn.relu(
        buv_r[...]
        + jnp.dot(diuv * (agg_r[...] + xsd_r[...]), wuv_r[...],
                  preferred_element_type=jnp.float32))
    xse_r[...] = jnp.concatenate([diu * x4, div * x4], axis=1)


def _t6_body(agg_r, xse_r, h_r, wu_r, bu_r, wv_r, bv_r, xsf_r):
    diu, div, diuv = _dinvs(h_r)
    tu = bu_r[...] + jnp.dot(diu * (agg_r[:, 0:32] + xse_r[:, 0:32]),
                             wu_r[...], preferred_element_type=jnp.float32)
    tv = bv_r[...] + jnp.dot(div * (agg_r[:, 32:64] + xse_r[:, 32:64]),
                             wv_r[...], preferred_element_type=jnp.float32)
    xsf_r[...] = diuv * jnp.where(_row_mask(128), tu, tv)


def _t7_body(agg_r, xsf_r, h_r, wuv_r, buv_r, out_r):
    _, _, diuv = _dinvs(h_r)
    out_r[...] = buv_r[...] + jnp.dot(
        diuv * (agg_r[...] + xsf_r[...]), wuv_r[...],
        preferred_element_type=jnp.float32)


def kernel(x, edge_index, edge_index_u, edge_index_v, params):
    f32 = jnp.float32
    ((Wu1, bu1, Wv1, bv1, Wuv1, buv1),
     (Wu2, bu2, Wv2, bv2, Wuv2, buv2),
     (Wu3, bu3, Wv3, bv3, Wuv3, buv3)) = params

    def pad_e(a, val):
        return jnp.pad(a, (0, _EP - a.shape[0]), constant_values=val)

    srcs = jnp.stack([pad_e(edge_index_u[0], 0), pad_e(edge_index_v[0], 0),
                      pad_e(edge_index[0], 0)]).reshape(3, _CPB, _K)
    dsts = jnp.stack([pad_e(edge_index_u[1], _N), pad_e(edge_index_v[1], _N),
                      pad_e(edge_index[1], _N)]).reshape(3, _CPB, _K)
    sd3 = jnp.stack([srcs, dsts], axis=2)

    ones16 = jnp.ones((_K, 16), f32)
    z16 = jnp.zeros((_K, 16), f32)
    z32 = jnp.zeros((_K, 32), f32)

    seg16 = _make_seg_sum(2, 16)
    seg16s = _make_seg_sum(2, 16, shared=True)
    seg32 = _make_seg_sum(2, 32)
    seg32s = _make_seg_sum(2, 32, shared=True)
    seg32x4s = _make_seg_sum(4, 32, shared=True)

    h = _make_hist()(ones16, sd3, z16).reshape(_NP, 64)
    xpad = jnp.pad(x, ((0, _NP - _N), (0, 0)))
    xsa = _tc_run(_t1_body, [xpad, h], [], [32])[0]

    agg_a = seg16(xsa.reshape(2 * _NP, 16), sd3, z16).reshape(_NP, 32)
    xsb = _tc_run(_t2_body, [agg_a, xsa, h],
                  [Wu1.reshape(1, 64), bu1.reshape(1, 64),
                   Wv1.reshape(1, 64), bv1.reshape(1, 64)], [64])[0]
    agg_b = seg32s(xsb.reshape(2 * _NP, 32), sd3, z32).reshape(_NP, 64)

    xsc = _tc_run(_t3_body, [agg_b, xsb, h],
                  [Wuv1, buv1.reshape(1, 64), Wu2, Wv2], [64])[0]
    agg_c = seg32(xsc.reshape(2 * _NP, 32), sd3, z32).reshape(_NP, 64)
    xsd = _tc_run(_t4_body, [agg_c, xsc, h],
                  [bu2.reshape(1, 32), bv2.reshape(1, 32)], [32])[0]
    agg_d = seg16s(xsd.reshape(2 * _NP, 16), sd3, z16).reshape(_NP, 32)

    xse = _tc_run(_t5_body, [agg_d, xsd, h],
                  [Wuv2, buv2.reshape(1, 32)], [64])[0]
    agg_e = seg32(xse.reshape(2 * _NP, 32), sd3, z32).reshape(_NP, 64)
    xsf = _tc_run(_t6_body, [agg_e, xse, h],
                  [Wu3, bu3.reshape(1, 128), Wv3, bv3.reshape(1, 128)],
                  [128])[0]
    agg_f = seg32x4s(xsf.reshape(4 * _NP, 32), sd3, z32).reshape(_NP, 128)
    out = _tc_run(_t7_body, [agg_f, xsf, h],
                  [Wuv3, buv3.reshape(1, 128)], [128])[0]
    return out[:_N]


# speedup vs baseline: 1.0075x; 1.0075x over previous
import functools

import jax
import jax.numpy as jnp
from jax import lax
from jax.experimental import pallas as pl
from jax.experimental.pallas import tpu as pltpu
from jax.experimental.pallas import tpu_sc as plsc

_N = 50000
_NU = 25000
_E = 800000
_NP = 51200
_K = 128
_GB = 4
_NSUB = 16
_NCORE = 2
_NRR = _NP // _NSUB
_BN = 3200


def _ceil_to(x, m):
    return (x + m - 1) // m * m


_EP = _ceil_to(_E, _K * _GB * _NSUB)
_CPB = _EP // _K


@functools.lru_cache(None)
def _make_seg_sum(nb, w, shared=False):
    gb = _GB if w == 16 else 2
    ch = nb // _NCORE
    cpt = _CPB // _NSUB
    ngrp = cpt // gb
    mesh = plsc.VectorSubcoreMesh(core_axis_name="c", subcore_axis_name="s")

    def body(vals, sd, zrow, out, acc, sdix, rows, gsem, ssem):
        c = lax.axis_index("c")
        s = lax.axis_index("s")

        def load(b, g, slot):
            sdi = 2 if shared else b
            pltpu.sync_copy(sd.at[sdi, pl.ds(s * cpt + g * gb, gb)],
                            sdix.at[slot])
            for j in range(gb):
                for t in range(_K // 16):
                    ix = pl.ds(t * 16, 16)
                    sdix[slot, j, 0, ix] = sdix[slot, j, 0, ix] * nb + b

        def fire_gathers(slot):
            for j in range(gb):
                pltpu.async_copy(vals.at[sdix.at[slot, j, 0]],
                                 rows.at[slot, j], gsem)

        def wait_gathers(slot):
            for j in range(gb):
                pltpu.make_async_copy(vals.at[sdix.at[slot, j, 0]],
                                      rows.at[slot, j], gsem).wait()

        def scatters(slot):
            descs = [
                pltpu.async_copy(rows.at[slot, j], acc.at[sdix.at[slot, j, 1]],
                                 ssem, add=True)
                for j in range(gb)
            ]
            for d in descs:
                d.wait()

        def step(b, g, p, prefetch, gnext):
            pn = 1 - p
            if prefetch:
                load(b, gnext, pn)
            wait_gathers(p)
            if prefetch:
                fire_gathers(pn)
            scatters(p)

        for bi in range(ch):
            b = c * ch + bi
            pltpu.sync_copy(zrow, rows.at[0, 0])
            zd = [pltpu.async_copy(rows.at[0, 0],
                                   acc.at[pl.ds(s * _NRR + r * _K, _K)], gsem)
                  for r in range(_NRR // _K)]
            for d in zd:
                d.wait()
            plsc.subcore_barrier()

            load(b, 0, 0)
            fire_gathers(0)
            m = (ngrp - 1) // 2

            def dbl(t, carry):
                g = t * 2
                step(b, g, 0, True, g + 1)
                step(b, g + 1, 1, True, g + 2)
                return carry

            lax.fori_loop(0, m, dbl, 0)
            g0 = 2 * m
            if (ngrp - 1) % 2 == 1:
                step(b, g0, 0, True, g0 + 1)
                step(b, g0 + 1, 1, False, 0)
            else:
                step(b, g0, 0, False, 0)
            plsc.subcore_barrier()
            pltpu.sync_copy(acc.at[pl.ds(s * _NRR, _NRR)],
                            out.at[pl.ds(s * _NRR, _NRR), b])
            plsc.subcore_barrier()

    return pl.kernel(
        body,
        out_type=jax.ShapeDtypeStruct((_NP, nb, w), jnp.float32),
        mesh=mesh,
        scratch_types=[
            pltpu.VMEM_SHARED((_NP, w), jnp.float32),
            pltpu.VMEM((2, gb, 2, _K), jnp.int32),
            pltpu.VMEM((2, gb, _K, w), jnp.float32),
            pltpu.SemaphoreType.DMA,
            pltpu.SemaphoreType.DMA,
        ],
        compiler_params=pltpu.CompilerParams(use_tc_tiling_on_sc=False),
    )


@functools.lru_cache(None)
def _make_hist():
    gb = _GB
    cpt_f = _CPB // _NSUB
    cpt_h = _CPB // 2 // _NSUB
    ngrp_f = cpt_f // gb
    ngrp_h = cpt_h // gb
    mesh = plsc.VectorSubcoreMesh(core_axis_name="c", subcore_axis_name="s")

    def body(ones, sd, zrow, out, acc, sdix, rows, gsem, ssem):
        c = lax.axis_index("c")
        s = lax.axis_index("s")
        for j in range(gb):
            pltpu.sync_copy(ones, rows.at[0, j])
        pltpu.sync_copy(zrow, rows.at[1, 0])

        for bi in range(2):
            b = c * 2 + bi
            sdi = c if bi == 0 else 2
            base0 = (s * cpt_f if bi == 0
                     else c * (_CPB // 2) + s * cpt_h)
            ngrp = ngrp_f if bi == 0 else ngrp_h
            zd = [pltpu.async_copy(rows.at[1, 0],
                                   acc.at[pl.ds(s * _NRR + r * _K, _K)], gsem)
                  for r in range(_NRR // _K)]
            for d in zd:
                d.wait()
            plsc.subcore_barrier()

            def grp(g, carry):
                pltpu.sync_copy(sd.at[sdi, pl.ds(base0 + g * gb, gb)],
                                sdix.at[0])
                descs = [
                    pltpu.async_copy(rows.at[0, j],
                                     acc.at[sdix.at[0, j, 1]], ssem, add=True)
                    for j in range(gb)
                ]
                for d in descs:
                    d.wait()
                return carry

            lax.fori_loop(0, ngrp, grp, 0)
            plsc.subcore_barrier()
            pltpu.sync_copy(acc.at[pl.ds(s * _NRR, _NRR)],
                            out.at[pl.ds(s * _NRR, _NRR), b])
            plsc.subcore_barrier()

    return pl.kernel(
        body,
        out_type=jax.ShapeDtypeStruct((_NP, 4, 16), jnp.float32),
        mesh=mesh,
        scratch_types=[
            pltpu.VMEM_SHARED((_NP, 16), jnp.float32),
            pltpu.VMEM((2, _GB, 2, _K), jnp.int32),
            pltpu.VMEM((2, _GB, _K, 16), jnp.float32),
            pltpu.SemaphoreType.DMA,
            pltpu.SemaphoreType.DMA,
        ],
        compiler_params=pltpu.CompilerParams(use_tc_tiling_on_sc=False),
    )


def _tc_run(body, row_ins, bcast_ins, out_cols):
    grid = (_NP // _BN,)
    in_specs = (
        [pl.BlockSpec((_BN, a.shape[1]), lambda i: (i, 0)) for a in row_ins]
        + [pl.BlockSpec(w.shape, lambda i: (0, 0)) for w in bcast_ins]
    )
    out_specs = [pl.BlockSpec((_BN, c), lambda i: (i, 0)) for c in out_cols]
    out_shape = [jax.ShapeDtypeStruct((_NP, c), jnp.float32) for c in out_cols]
    return pl.pallas_call(
        body, grid=grid, in_specs=in_specs, out_specs=out_specs,
        out_shape=out_shape,
    )(*row_ins, *bcast_ins)


def _row_mask(cols):
    rid = (pl.program_id(0) * _BN
           + lax.broadcasted_iota(jnp.int32, (_BN, cols), 0))
    return rid < _NU


def _dinvs(h_r):
    diu = 1.0 / jnp.sqrt(h_r[:, 0:1] + 1.0)
    div = 1.0 / jnp.sqrt(h_r[:, 32:33] + 1.0)
    diuv = 1.0 / jnp.sqrt(h_r[:, 16:17] + h_r[:, 48:49] + 1.0)
    return diu, div, diuv


def _t1_body(x_r, h_r, xsa_r):
    diu, div, _ = _dinvs(h_r)
    x = x_r[...]
    xsa_r[...] = jnp.concatenate(
        [jnp.broadcast_to(x * diu, (_BN, 16)),
         jnp.broadcast_to(x * div, (_BN, 16))], axis=1)


def _t2_body(agg_r, xsa_r, h_r, wu_r, bu_r, wv_r, bv_r, xsb_r):
    diu, div, diuv = _dinvs(h_r)
    aggu0 = agg_r[:, 0:1] + xsa_r[:, 0:1]
    aggv0 = agg_r[:, 16:17] + xsa_r[:, 16:17]
    yu = bu_r[...] + (diu * aggu0) * wu_r[...]
    yv = bv_r[...] + (div * aggv0) * wv_r[...]
    xsb_r[...] = diuv * jnp.where(_row_mask(64), yu, yv)


def _t3_body(agg_r, xsb_r, h_r, wuv_r, buv_r, wu2_r, wv2_r, xsc_r):
    diu, div, diuv = _dinvs(h_r)
    x2 = jax.nn.relu(
        buv_r[...]
        + jnp.dot(diuv * (agg_r[...] + xsb_r[...]), wuv_r[...],
                  preferred_element_type=jnp.float32))
    xsc_r[...] = jnp.concatenate(
        [diu * jnp.dot(x2, wu2_r[...], preferred_element_type=jnp.float32),
         div * jnp.dot(x2, wv2_r[...], preferred_element_type=jnp.float32)],
        axis=1)


def _t4_body(agg_r, xsc_r, h_r, bu_r, bv_r, xsd_r):
    diu, div, diuv = _dinvs(h_r)
    outu = bu_r[...] + diu * (agg_r[:, 0:32] + xsc_r[:, 0:32])
    outv = bv_r[...] + div * (agg_r[:, 32:64] + xsc_r[:, 32:64])
    xsd_r[...] = diuv * jnp.where(_row_mask(32), outu, outv)


def _t5_body(agg_r, xsd_r, h_r, wuv_r, buv_r, xse_r):
    diu, div, diuv = _dinvs(h_r)
    x4 = jax.nn.relu(
        buv_r[...]
        + jnp.dot(diuv * (agg_r[...] + xsd_r[...]), wuv_r[...],
                  preferred_element_type=jnp.float32))
    xse_r[...] = jnp.concatenate([diu * x4, div * x4], axis=1)


def _t6_body(agg_r, xse_r, h_r, wu_r, bu_r, wv_r, bv_r, xsf_r):
    diu, div, diuv = _dinvs(h_r)
    tu = bu_r[...] + jnp.dot(diu * (agg_r[:, 0:32] + xse_r[:, 0:32]),
                             wu_r[...], preferred_element_type=jnp.float32)
    tv = bv_r[...] + jnp.dot(div * (agg_r[:, 32:64] + xse_r[:, 32:64]),
                             wv_r[...], preferred_element_type=jnp.float32)
    xsf_r[...] = diuv * jnp.where(_row_mask(128), tu, tv)


def _t7_body(agg_r, xsf_r, h_r, wuv_r, buv_r, out_r):
    _, _, diuv = _dinvs(h_r)
    out_r[...] = buv_r[...] + jnp.dot(
        diuv * (agg_r[...] + xsf_r[...]), wuv_r[...],
        preferred_element_type=jnp.float32)


def kernel(x, edge_index, edge_index_u, edge_index_v, params):
    f32 = jnp.float32
    ((Wu1, bu1, Wv1, bv1, Wuv1, buv1),
     (Wu2, bu2, Wv2, bv2, Wuv2, buv2),
     (Wu3, bu3, Wv3, bv3, Wuv3, buv3)) = params

    def pad_e(a, val):
        return jnp.pad(a, (0, _EP - a.shape[0]), constant_values=val)

    srcs = jnp.stack([pad_e(edge_index_u[0], 0), pad_e(edge_index_v[0], 0),
                      pad_e(edge_index[0], 0)]).reshape(3, _CPB, _K)
    dsts = jnp.stack([pad_e(edge_index_u[1], _N), pad_e(edge_index_v[1], _N),
                      pad_e(edge_index[1], _N)]).reshape(3, _CPB, _K)
    sd3 = jnp.stack([srcs, dsts], axis=2)

    ones16 = jnp.ones((_K, 16), f32)
    z16 = jnp.zeros((_K, 16), f32)
    z32 = jnp.zeros((_K, 32), f32)

    seg16 = _make_seg_sum(2, 16)
    seg16s = _make_seg_sum(2, 16, shared=True)
    seg32 = _make_seg_sum(2, 32)
    seg32s = _make_seg_sum(2, 32, shared=True)
    seg32x4s = _make_seg_sum(4, 32, shared=True)

    h = _make_hist()(ones16, sd3, z16).reshape(_NP, 64)
    xpad = jnp.pad(x, ((0, _NP - _N), (0, 0)))
    xsa = _tc_run(_t1_body, [xpad, h], [], [32])[0]

    agg_a = seg16(xsa.reshape(2 * _NP, 16), sd3, z16).reshape(_NP, 32)
    xsb = _tc_run(_t2_body, [agg_a, xsa, h],
                  [Wu1.reshape(1, 64), bu1.reshape(1, 64),
                   Wv1.reshape(1, 64), bv1.reshape(1, 64)], [64])[0]
    agg_b = seg32s(xsb.reshape(2 * _NP, 32), sd3, z32).reshape(_NP, 64)

    xsc = _tc_run(_t3_body, [agg_b, xsb, h],
                  [Wuv1, buv1.reshape(1, 64), Wu2, Wv2], [64])[0]
    agg_c = seg32(xsc.reshape(2 * _NP, 32), sd3, z32).reshape(_NP, 64)
    xsd = _tc_run(_t4_body, [agg_c, xsc, h],
                  [bu2.reshape(1, 32), bv2.reshape(1, 32)], [32])[0]
    agg_d = seg16s(xsd.reshape(2 * _NP, 16), sd3, z16).reshape(_NP, 32)

    xse = _tc_run(_t5_body, [agg_d, xsd, h],
                  [Wuv2, buv2.reshape(1, 32)], [64])[0]
    agg_e = seg32(xse.reshape(2 * _NP, 32), sd3, z32).reshape(_NP, 64)
    xsf = _tc_run(_t6_body, [agg_e, xse, h],
                  [Wu3, bu3.reshape(1, 128), Wv3, bv3.reshape(1, 128)],
                  [128])[0]
    agg_f = seg32x4s(xsf.reshape(4 * _NP, 32), sd3, z32).reshape(_NP, 128)
    out = _tc_run(_t7_body, [agg_f, xsf, h],
                  [Wuv3, buv3.reshape(1, 128)], [128])[0]
    return out[:_N]
